# baseline (speedup 1.0000x reference)
import jax
import jax.numpy as jnp
import numpy as np
from jax import lax
from jax.experimental import pallas as pl
from jax.experimental.pallas import tpu as pltpu
from jax.experimental.pallas import tpu_sc as plsc

ROWS = 2048
COLS = 2048
N = ROWS * COLS
NHEAD = 16
MAX_LEN = 1001
PLANE = 1008
GRID = np.float32(0.001)
NC, NS, L = 2, 16, 16
NW = NC * NS
PER_W = N // NW
ROWS_W = PER_W // COLS
CHUNK = 256
GROUPS = CHUNK // L
STEP_EL = 2 * CHUNK
STEPS = PER_W // STEP_EL
SPR = COLS // STEP_EL
HIMASK = np.int32(-65536)


def _body(rel0_hbm, rel1_hbm, t1_hbm, t2_hbm, out_hbm,
          t1v, t2v, r0v, r1v, outA, outB, semA, semB, semI):
    wid = lax.axis_index("s") * NC + lax.axis_index("c")
    iota = lax.iota(jnp.int32, L)
    pltpu.sync_copy(t1_hbm, t1v)
    pltpu.sync_copy(t2_hbm, t2v)
    row0 = wid * ROWS_W
    pltpu.async_copy(rel0_hbm.at[row0], r0v.at[0], semI)
    pltpu.async_copy(rel1_hbm.at[row0], r1v.at[0], semI)

    def step(s, carry):
        rloc = lax.div(s, SPR)
        row = row0 + rloc
        par = lax.rem(rloc, 2)
        c_base = lax.rem(s, SPR) * STEP_EL
        par_v = jnp.full((L,), 0, jnp.int32) + par

        @pl.when(c_base == 0)
        def _():
            pltpu.make_async_copy(rel0_hbm.at[row0], r0v.at[0], semI).wait()
            pltpu.make_async_copy(rel1_hbm.at[row0], r1v.at[0], semI).wait()

        @pl.when((lax.rem(s, SPR) == SPR - 1) & (s < STEPS - 1))
        def _():
            pltpu.async_copy(rel0_hbm.at[row + 1], r0v.at[1 - par], semI)
            pltpu.async_copy(rel1_hbm.at[row + 1], r1v.at[1 - par], semI)

        for half, outv, sem in ((0, outA, semA), (1, outB, semB)):
            cb = c_base + half * CHUNK
            cb_v = iota + cb

            @pl.when(s > 0)
            def _():
                pltpu.make_async_copy(
                    outv, out_hbm.at[0, :, pl.ds(0, CHUNK)], sem).wait()

            @plsc.parallel_loop(0, GROUPS, 1, unroll=2)
            def grp(g):
                e_idx = cb_v + g * L
                c0 = plsc.load_gather(r0v, [par_v, e_idx])
                c1 = plsc.load_gather(r1v, [par_v, e_idx])
                dist = c0 / GRID
                i1 = dist.astype(jnp.int32)
                frac = dist - i1.astype(jnp.float32)
                dist_t = c1 / GRID
                j1 = dist_t.astype(jnp.int32)
                w2t = dist - j1.astype(jnp.float32)
                ia = jnp.minimum(i1, MAX_LEN - 1)
                jb = jnp.minimum(j1, MAX_LEN - 1)
                for d in range(NHEAD):
                    wa = plsc.load_gather(t1v, [ia + d * PLANE])
                    wb = plsc.load_gather(t2v, [jb + d * PLANE])
                    a1 = plsc.bitcast(jnp.left_shift(wa, 16), jnp.float32)
                    da = plsc.bitcast(jnp.bitwise_and(wa, HIMASK), jnp.float32)
                    b1 = plsc.bitcast(jnp.left_shift(wb, 16), jnp.float32)
                    db = plsc.bitcast(jnp.bitwise_and(wb, HIMASK), jnp.float32)
                    es = a1 + frac * da
                    et = b1 + w2t * db
                    outv[d, pl.ds(g * L, L)] = es * et
                return None

            pltpu.async_copy(
                outv, out_hbm.at[row, :, pl.ds(cb, CHUNK)], sem)
        return carry

    lax.fori_loop(0, STEPS, step, 0)
    pltpu.make_async_copy(
        outA, out_hbm.at[0, :, pl.ds(0, CHUNK)], semA).wait()
    pltpu.make_async_copy(
        outB, out_hbm.at[0, :, pl.ds(0, CHUNK)], semB).wait()


def _pack(table):
    diff = jnp.concatenate([table[1:], table[-1:]], axis=0) - table
    lo = lax.bitcast_convert_type(
        table.astype(jnp.bfloat16), jnp.uint16).astype(jnp.uint32)
    hi = lax.bitcast_convert_type(
        diff.astype(jnp.bfloat16), jnp.uint16).astype(jnp.uint32)
    packed = jnp.bitwise_or(lo, jnp.left_shift(hi, 16))
    packed = jnp.transpose(packed, (1, 0))
    packed = jnp.pad(packed, ((0, 0), (0, PLANE - MAX_LEN)))
    return lax.bitcast_convert_type(packed, jnp.int32).reshape(-1)


def kernel(rel_pos, pos_embed, pos_embed_t):
    rel0 = rel_pos[0, :, :, 0]
    rel1 = rel_pos[0, :, :, 1]
    t1 = _pack(pos_embed)
    t2 = _pack(pos_embed_t)
    mesh = plsc.VectorSubcoreMesh(core_axis_name="c", subcore_axis_name="s")
    out = pl.kernel(
        _body,
        mesh=mesh,
        compiler_params=pltpu.CompilerParams(needs_layout_passes=False),
        out_type=jax.ShapeDtypeStruct((ROWS, NHEAD, COLS), jnp.float32),
        scratch_types=[
            pltpu.VMEM((NHEAD * PLANE,), jnp.int32),
            pltpu.VMEM((NHEAD * PLANE,), jnp.int32),
            pltpu.VMEM((2, COLS), jnp.float32),
            pltpu.VMEM((2, COLS), jnp.float32),
            pltpu.VMEM((NHEAD, CHUNK), jnp.float32),
            pltpu.VMEM((NHEAD, CHUNK), jnp.float32),
            pltpu.SemaphoreType.DMA,
            pltpu.SemaphoreType.DMA,
            pltpu.SemaphoreType.DMA,
        ],
    )(rel0, rel1, t1, t2)
    return jnp.transpose(out, (0, 2, 1)).reshape(1, ROWS, COLS, NHEAD)


# speedup vs baseline: 1.4642x; 1.4642x over previous
import jax
import jax.numpy as jnp
import numpy as np
from jax import lax
from jax.experimental import pallas as pl
from jax.experimental.pallas import tpu as pltpu
from jax.experimental.pallas import tpu_sc as plsc

ROWS = 2048
COLS = 2048
N = ROWS * COLS
NHEAD = 16
MAX_LEN = 1001
PLANE = 1008
GRID = np.float32(0.001)
NC, NS, L = 2, 16, 16
NW = NC * NS
PER_W = N // NW
ROWS_W = PER_W // COLS
CHUNK = 128
GROUPS = CHUNK // L
STEP_EL = 2 * CHUNK
STEPS = PER_W // STEP_EL
SPR = COLS // STEP_EL
HIMASK = np.int32(-65536)


def _body(rel0_hbm, rel1_hbm, t1_hbm, t2_hbm, out_hbm,
          t1v, t2v, r0v, r1v, outA, outB, semA, semB, semI):
    wid = lax.axis_index("s") * NC + lax.axis_index("c")
    iota = lax.iota(jnp.int32, L)
    pltpu.sync_copy(t1_hbm, t1v)
    pltpu.sync_copy(t2_hbm, t2v)
    row0 = wid * ROWS_W
    pltpu.async_copy(rel0_hbm.at[row0], r0v.at[0], semI)
    pltpu.async_copy(rel1_hbm.at[row0], r1v.at[0], semI)

    def step(s, carry):
        rloc = lax.div(s, SPR)
        row = row0 + rloc
        par = lax.rem(rloc, 2)
        c_base = lax.rem(s, SPR) * STEP_EL
        par_v = jnp.full((L,), 0, jnp.int32) + par

        @pl.when(c_base == 0)
        def _():
            pltpu.make_async_copy(rel0_hbm.at[row0], r0v.at[0], semI).wait()
            pltpu.make_async_copy(rel1_hbm.at[row0], r1v.at[0], semI).wait()

        @pl.when((lax.rem(s, SPR) == SPR - 1) & (s < STEPS - 1))
        def _():
            pltpu.async_copy(rel0_hbm.at[row + 1], r0v.at[1 - par], semI)
            pltpu.async_copy(rel1_hbm.at[row + 1], r1v.at[1 - par], semI)

        for half, outv, sem in ((0, outA, semA), (1, outB, semB)):
            cb = c_base + half * CHUNK
            cb_v = iota + cb

            @pl.when(s > 0)
            def _():
                pltpu.make_async_copy(
                    outv, out_hbm.at[0, :, pl.ds(0, CHUNK)], sem).wait()

            @plsc.parallel_loop(0, GROUPS, 1, unroll=2)
            def grp(g):
                e_idx = cb_v + g * L
                c0 = plsc.load_gather(r0v, [par_v, e_idx])
                c1 = plsc.load_gather(r1v, [par_v, e_idx])
                dist = c0 / GRID
                i1 = dist.astype(jnp.int32)
                frac = dist - i1.astype(jnp.float32)
                dist_t = c1 / GRID
                j1 = dist_t.astype(jnp.int32)
                w2t = dist - j1.astype(jnp.float32)
                ia = jnp.minimum(i1, MAX_LEN - 1)
                jb = jnp.minimum(j1, MAX_LEN - 1)
                for d in range(NHEAD):
                    wa = plsc.load_gather(t1v, [ia + d * PLANE])
                    wb = plsc.load_gather(t2v, [jb + d * PLANE])
                    a1 = plsc.bitcast(jnp.left_shift(wa, 16), jnp.float32)
                    da = plsc.bitcast(jnp.bitwise_and(wa, HIMASK), jnp.float32)
                    b1 = plsc.bitcast(jnp.left_shift(wb, 16), jnp.float32)
                    db = plsc.bitcast(jnp.bitwise_and(wb, HIMASK), jnp.float32)
                    es = a1 + frac * da
                    et = b1 + w2t * db
                    outv[d, pl.ds(g * L, L)] = es * et
                return None

            pltpu.async_copy(
                outv, out_hbm.at[row, :, pl.ds(cb, CHUNK)], sem)
        return carry

    lax.fori_loop(0, STEPS, step, 0)
    pltpu.make_async_copy(
        outA, out_hbm.at[0, :, pl.ds(0, CHUNK)], semA).wait()
    pltpu.make_async_copy(
        outB, out_hbm.at[0, :, pl.ds(0, CHUNK)], semB).wait()


def _pack(table):
    diff = jnp.concatenate([table[1:], table[-1:]], axis=0) - table
    lo = lax.bitcast_convert_type(
        table.astype(jnp.bfloat16), jnp.uint16).astype(jnp.uint32)
    hi = lax.bitcast_convert_type(
        diff.astype(jnp.bfloat16), jnp.uint16).astype(jnp.uint32)
    packed = jnp.bitwise_or(lo, jnp.left_shift(hi, 16))
    packed = jnp.transpose(packed, (1, 0))
    packed = jnp.pad(packed, ((0, 0), (0, PLANE - MAX_LEN)))
    return lax.bitcast_convert_type(packed, jnp.int32).reshape(-1)


def kernel(rel_pos, pos_embed, pos_embed_t):
    rel0 = rel_pos[0, :, :, 0]
    rel1 = rel_pos[0, :, :, 1]
    t1 = _pack(pos_embed)
    t2 = _pack(pos_embed_t)
    mesh = plsc.VectorSubcoreMesh(core_axis_name="c", subcore_axis_name="s")
    out = pl.kernel(
        _body,
        mesh=mesh,
        compiler_params=pltpu.CompilerParams(needs_layout_passes=False),
        out_type=jax.ShapeDtypeStruct((ROWS, NHEAD, COLS), jnp.float32),
        scratch_types=[
            pltpu.VMEM((NHEAD * PLANE,), jnp.int32),
            pltpu.VMEM((NHEAD * PLANE,), jnp.int32),
            pltpu.VMEM((2, COLS), jnp.float32),
            pltpu.VMEM((2, COLS), jnp.float32),
            pltpu.VMEM((NHEAD, CHUNK), jnp.float32),
            pltpu.VMEM((NHEAD, CHUNK), jnp.float32),
            pltpu.SemaphoreType.DMA,
            pltpu.SemaphoreType.DMA,
            pltpu.SemaphoreType.DMA,
        ],
    )(rel0, rel1, t1, t2)
    return jnp.transpose(out, (0, 2, 1)).reshape(1, ROWS, COLS, NHEAD)
